# baseline (speedup 1.0000x reference)
import jax
import jax.numpy as jnp
from jax import lax
from jax.experimental import pallas as pl
from jax.experimental.pallas import tpu as pltpu
from jax.experimental.pallas import tpu_sc as plsc

_B = 16384
_V = 27
_D = 27 * 27
_P = 736
_NC = 2
_NS = 16
_NW = _NC * _NS
_BPW = _B // _NW
_G = 16
_GW = _G * _D
_GPC = 4
_CW = _GPC * _GW
_NCH = _BPW // (_G * _GPC)


def _sc_body(x_hbm, w_hbm, out_hbm, table_v, pack_v, idx_v, sem):
    wid = lax.axis_index("s") * _NC + lax.axis_index("c")
    base = wid * _BPW
    pltpu.sync_copy(w_hbm, table_v)
    pltpu.sync_copy(x_hbm.at[pl.ds(base, _BPW)], idx_v)
    iota = lax.iota(jnp.int32, 16)

    def group_body(g, _):
        pbase = (g % _GPC) * _GW
        g16 = g * _G
        for w in range(_GW // 16):
            lo = 16 * w
            ri0 = lo // _D
            ri1 = (lo + 15) // _D
            if ri0 == ri1:
                pat = jnp.broadcast_to(g16 + ri0, (16,))
                off = iota + (lo - _D * ri0)
            else:
                split = _D * ri1 - lo
                pat = jnp.where(iota < split, g16 + ri0, g16 + ri1)
                off = iota + jnp.where(
                    iota < split, lo - _D * ri0, lo - _D * ri1
                )
            xi = plsc.load_gather(idx_v, [pat])
            pack_v[pl.ds(pbase + lo, 16)] = plsc.load_gather(
                table_v, [xi * _P + off]
            )
        return _

    def chunk_body(ci, _):
        lax.fori_loop(ci * _GPC, (ci + 1) * _GPC, group_body, 0)
        pltpu.async_copy(
            pack_v.at[pl.ds(0, _CW)],
            out_hbm.at[pl.ds(base * _D + ci * _CW, _CW)],
            sem,
        ).wait()
        return _

    lax.fori_loop(0, _NCH, chunk_body, 0)


@jax.jit
def kernel(x, W):
    x32 = x.astype(jnp.int32)
    w1d = jnp.pad(W.reshape(_V, _D), ((0, 0), (0, _P - _D))).reshape(_V * _P)
    mesh = plsc.VectorSubcoreMesh(core_axis_name="c", subcore_axis_name="s")
    out = pl.kernel(
        _sc_body,
        out_type=jax.ShapeDtypeStruct((_B * _D,), jnp.float32),
        mesh=mesh,
        compiler_params=pltpu.CompilerParams(needs_layout_passes=False),
        scratch_types=[
            pltpu.VMEM((_V * _P,), jnp.float32),
            pltpu.VMEM((_CW,), jnp.float32),
            pltpu.VMEM((_BPW,), jnp.int32),
            pltpu.SemaphoreType.DMA,
        ],
    )(x32, w1d)
    return out.reshape(_B, 27, 27)


# speedup vs baseline: 5.1256x; 5.1256x over previous
import jax
import jax.numpy as jnp
from jax import lax
from jax.experimental import pallas as pl
from jax.experimental.pallas import tpu as pltpu
from jax.experimental.pallas import tpu_sc as plsc

_B = 16384
_V = 27
_D = 27 * 27
_NC = 2
_NS = 16
_NW = _NC * _NS
_BPW = _B // _NW
_K = 8


def _sc_body(x_hbm, w_hbm, out_hbm, table_v, idx_v, sem):
    wid = lax.axis_index("s") * _NC + lax.axis_index("c")
    base = wid * _BPW
    pltpu.sync_copy(w_hbm, table_v)
    pltpu.sync_copy(x_hbm.at[pl.ds(base, _BPW)], idx_v)

    def group_body(g, _):
        vec = idx_v[pl.ds(g * 16, 16)]
        for l in range(16):
            pltpu.async_copy(
                table_v.at[vec[l]], out_hbm.at[base + g * 16 + l], sem
            )

        @pl.when(g >= 1)
        def _drain():
            for _u in range(16):
                pltpu.make_async_copy(w_hbm.at[0], table_v.at[0], sem).wait()

        return _

    lax.fori_loop(0, _BPW // 16, group_body, 0)
    for _u in range(16):
        pltpu.make_async_copy(w_hbm.at[0], table_v.at[0], sem).wait()


@jax.jit
def kernel(x, W):
    x32 = x.astype(jnp.int32)
    w2d = W.reshape(_V, _D)
    mesh = plsc.VectorSubcoreMesh(core_axis_name="c", subcore_axis_name="s")
    out = pl.kernel(
        _sc_body,
        out_type=jax.ShapeDtypeStruct((_B, _D), jnp.float32),
        mesh=mesh,
        compiler_params=pltpu.CompilerParams(needs_layout_passes=False),
        scratch_types=[
            pltpu.VMEM((_V, _D), jnp.float32),
            pltpu.VMEM((_BPW,), jnp.int32),
            pltpu.SemaphoreType.DMA,
        ],
    )(x32, w2d)
    return out.reshape(_B, 27, 27)
